# Initial kernel scaffold; baseline (speedup 1.0000x reference)
import jax
import jax.numpy as jnp
from jax.experimental import pallas as pl


def kernel(x, W1, b1, W2, b2, W3, b3, Wf1, bf1, Wf2, bf2, Wf3, bf3):
    raise NotImplementedError("write your pallas kernel here")


# speedup vs baseline: 8.6020x; 8.6020x over previous
import functools

import jax
import jax.numpy as jnp
from jax import lax
from jax.experimental import pallas as pl
from jax.experimental.pallas import tpu as pltpu
from jax.experimental.pallas import tpu_sc as plsc

KNN = 20


def _knn_proj_body(x_rows_ref, xt_ref, wpq_ref, bpq_ref,
                   idx_ref, p_ref, q_ref, *, n, k, rows):
    b = pl.program_id(0)
    xr = x_rows_ref[0]
    xt = xt_ref[0]
    sq_r = jnp.sum(xr * xr, axis=1, keepdims=True)
    sq_a = jnp.sum(xt * xt, axis=0, keepdims=True)
    inner = jnp.dot(xr, xt, preferred_element_type=jnp.float32)
    dist = (sq_r - 2.0 * inner) + sq_a

    iota = lax.broadcasted_iota(jnp.int32, (rows, n), 1)
    cur = dist
    cols = []
    for _ in range(k):
        m = jnp.min(cur, axis=1, keepdims=True)
        amin = jnp.min(jnp.where(cur == m, iota, n), axis=1,
                       keepdims=True)
        cur = jnp.where(iota == amin, jnp.inf, cur)
        cols.append(amin)
    idx = jnp.concatenate(cols, axis=1) + b * n
    idx_ref[0] = idx

    pq = jnp.dot(xr, wpq_ref[...], preferred_element_type=jnp.float32)
    pq = pq + bpq_ref[...]
    cout = pq.shape[1] // 2
    p_ref[0] = pq[:, :cout]
    q_ref[0] = pq[:, cout:]


def _knn_proj(x, wpq, bpq, *, rows=256):
    bsz, n, c = x.shape
    c2 = wpq.shape[1]
    cout = c2 // 2
    xt = jnp.swapaxes(x, 1, 2)
    grid = (bsz, n // rows)
    body = functools.partial(_knn_proj_body, n=n, k=KNN, rows=rows)
    return pl.pallas_call(
        body,
        grid=grid,
        in_specs=[
            pl.BlockSpec((1, rows, c), lambda b, r: (b, r, 0)),
            pl.BlockSpec((1, c, n), lambda b, r: (b, 0, 0)),
            pl.BlockSpec((c, c2), lambda b, r: (0, 0)),
            pl.BlockSpec((1, c2), lambda b, r: (0, 0)),
        ],
        out_specs=[
            pl.BlockSpec((1, rows, KNN), lambda b, r: (b, r, 0)),
            pl.BlockSpec((1, rows, cout), lambda b, r: (b, r, 0)),
            pl.BlockSpec((1, rows, cout), lambda b, r: (b, r, 0)),
        ],
        out_shape=[
            jax.ShapeDtypeStruct((bsz, n, KNN), jnp.int32),
            jax.ShapeDtypeStruct((bsz, n, cout), jnp.float32),
            jax.ShapeDtypeStruct((bsz, n, cout), jnp.float32),
        ],
    )(x, xt, wpq, bpq)


def _gather_max(p_rows, q_rows, idx_flat, *, pts_step=4):
    m, cout = q_rows.shape
    info = plsc.get_sparse_core_info()
    nw = info.num_cores * info.num_subcores
    pts_w = m // nw
    idx_step = pts_step * KNN
    steps = pts_w // pts_step
    mesh = plsc.VectorSubcoreMesh(core_axis_name="c", subcore_axis_name="s")

    @functools.partial(
        pl.kernel, mesh=mesh,
        out_type=jax.ShapeDtypeStruct((m, cout), jnp.float32),
        scratch_types=[
            pltpu.VMEM((idx_step,), jnp.int32),
            pltpu.VMEM((idx_step, cout), jnp.float32),
            pltpu.VMEM((pts_step, cout), jnp.float32),
            pltpu.VMEM((pts_step, cout), jnp.float32),
            pltpu.SemaphoreType.DMA,
        ],
    )
    def k(p_hbm, q_hbm, idx_hbm, out_hbm, idx_v, rows_v, p_v, o_v, sem):
        wid = lax.axis_index("s") * info.num_cores + lax.axis_index("c")

        def body(s, carry):
            base_pt = wid * pts_w + s * pts_step
            pltpu.sync_copy(idx_hbm.at[pl.ds(base_pt * KNN, idx_step)], idx_v)
            pltpu.async_copy(q_hbm.at[idx_v], rows_v, sem).wait()
            pltpu.sync_copy(p_hbm.at[pl.ds(base_pt, pts_step)], p_v)
            for p in range(pts_step):
                for l in range(cout // 16):
                    sl = pl.ds(l * 16, 16)
                    acc = rows_v[p * KNN, sl]
                    for kk in range(1, KNN):
                        acc = jnp.maximum(acc, rows_v[p * KNN + kk, sl])
                    o_v[p, sl] = acc + p_v[p, sl]
            pltpu.sync_copy(o_v, out_hbm.at[pl.ds(base_pt, pts_step)])
            return carry

        lax.fori_loop(0, steps, body, 0)

    return k(p_rows, q_rows, idx_flat)


def _head_body(h_ref, wf1_ref, bf1_ref, wf2_ref, bf2_ref, wf3_ref, bf3_ref,
               o_ref):
    g = jnp.max(h_ref[...], axis=1)
    z = jnp.dot(g, wf1_ref[...], preferred_element_type=jnp.float32)
    z = jnp.maximum(z + bf1_ref[...], 0.0)
    z = jnp.dot(z, wf2_ref[...], preferred_element_type=jnp.float32)
    z = jnp.maximum(z + bf2_ref[...], 0.0)
    z = jnp.dot(z, wf3_ref[...], preferred_element_type=jnp.float32)
    o_ref[...] = z + bf3_ref[...]


def _head(h, wf1, bf1, wf2, bf2, wf3, bf3):
    bsz, n, c = h.shape
    return pl.pallas_call(
        _head_body,
        out_shape=jax.ShapeDtypeStruct((bsz, wf3.shape[1]), jnp.float32),
    )(h, wf1, bf1, wf2, bf2, wf3, bf3)


def _edge_conv(h, w, b):
    bsz, n, c = h.shape
    cout = w.shape[1]
    wpq = jnp.concatenate([w[:c] - w[c:], w[c:]], axis=1)
    bpq = jnp.concatenate([b, jnp.zeros_like(b)])[None, :]
    idx, p_arr, q_arr = _knn_proj(h, wpq, bpq)
    p_rows = p_arr.reshape(bsz * n, cout)
    q_rows = q_arr.reshape(bsz * n, cout)
    cpad = max(cout, 128)
    if cpad != cout:
        p_rows = jnp.pad(p_rows, ((0, 0), (0, cpad - cout)))
        q_rows = jnp.pad(q_rows, ((0, 0), (0, cpad - cout)))
    out = _gather_max(p_rows, q_rows, idx.reshape(bsz * n * KNN))
    return out[:, :cout].reshape(bsz, n, cout)


def kernel(x, W1, b1, W2, b2, W3, b3, Wf1, bf1, Wf2, bf2, Wf3, bf3):
    h1 = _edge_conv(x, W1, b1)
    h2 = _edge_conv(h1, W2, b2)
    h3 = _edge_conv(h2, W3, b3)
    return _head(h3, Wf1, bf1[None, :], Wf2, bf2[None, :], Wf3, bf3[None, :])


# speedup vs baseline: 11.5789x; 1.3461x over previous
import functools

import jax
import jax.numpy as jnp
from jax import lax
from jax.experimental import pallas as pl
from jax.experimental.pallas import tpu as pltpu
from jax.experimental.pallas import tpu_sc as plsc

KNN = 20
CPAD = 128


def _knn_body(x_rows_ref, xt_ref, idx_ref, *, n, k, rows):
    b = pl.program_id(0)
    xr = x_rows_ref[0]
    xt = xt_ref[0]
    sq_r = jnp.sum(xr * xr, axis=1, keepdims=True)
    sq_a = jnp.sum(xt * xt, axis=0, keepdims=True)
    inner = jnp.dot(xr, xt, preferred_element_type=jnp.float32)
    dist = (sq_r - 2.0 * inner) + sq_a

    iota = lax.broadcasted_iota(jnp.int32, (rows, n), 1)
    cur = dist
    cols = []
    for _ in range(k):
        m = jnp.min(cur, axis=1, keepdims=True)
        amin = jnp.min(jnp.where(cur == m, iota, n), axis=1,
                       keepdims=True)
        cur = jnp.where(iota == amin, jnp.inf, cur)
        cols.append(amin)
    idx_ref[0] = jnp.concatenate(cols, axis=1) + b * n


def _knn(x, *, rows=256):
    bsz, n, c = x.shape
    xt = jnp.swapaxes(x, 1, 2)
    body = functools.partial(_knn_body, n=n, k=KNN, rows=rows)
    return pl.pallas_call(
        body,
        grid=(bsz, n // rows),
        in_specs=[
            pl.BlockSpec((1, rows, c), lambda b, r: (b, r, 0)),
            pl.BlockSpec((1, c, n), lambda b, r: (b, 0, 0)),
        ],
        out_specs=pl.BlockSpec((1, rows, KNN), lambda b, r: (b, r, 0)),
        out_shape=jax.ShapeDtypeStruct((bsz, n, KNN), jnp.int32),
    )(x, xt)


def _gather_rows(xpad, idx_wmajor):
    m = xpad.shape[0]
    info = plsc.get_sparse_core_info()
    nw = info.num_cores * info.num_subcores
    pts_w = m // nw
    half = pts_w // 2
    mesh = plsc.VectorSubcoreMesh(core_axis_name="c", subcore_axis_name="s")

    @functools.partial(
        pl.kernel, mesh=mesh,
        out_type=jax.ShapeDtypeStruct((KNN, m, CPAD), jnp.float32),
        scratch_types=[
            pltpu.VMEM((KNN * pts_w,), jnp.int32),
            pltpu.VMEM((half, CPAD), jnp.float32),
            pltpu.VMEM((half, CPAD), jnp.float32),
            pltpu.SemaphoreType.DMA,
            pltpu.SemaphoreType.DMA,
        ],
    )
    def k(x_hbm, idx_hbm, nb_hbm, idx_v, buf0, buf1, sem0, sem1):
        wid = lax.axis_index("s") * info.num_cores + lax.axis_index("c")
        base = wid * pts_w
        pltpu.sync_copy(idx_hbm.at[pl.ds(wid * (KNN * pts_w), KNN * pts_w)],
                        idx_v)
        bufs = ((buf0, sem0), (buf1, sem1))

        def fire(t, buf, sem):
            off = t * half
            for j in range(2):
                pltpu.async_copy(
                    x_hbm.at[idx_v.at[pl.ds(off + j * 128, 128)]],
                    buf.at[pl.ds(j * 128, 128), :], sem)

        fire(0, buf0, sem0)
        fire(1, buf1, sem1)

        def body(i, carry):
            for b in range(2):
                buf, sem = bufs[b]
                for j in range(2):
                    pltpu.make_async_copy(
                        x_hbm.at[idx_v.at[pl.ds(0, 128)]],
                        buf.at[pl.ds(j * 128, 128), :], sem).wait()
                pltpu.sync_copy(buf,
                                nb_hbm.at[i, pl.ds(base + b * half, half), :])

                @pl.when(i < KNN - 1)
                def _():
                    fire(2 * i + b + 2, buf, sem)

            return carry

        lax.fori_loop(0, KNN, body, 0)

    return k(xpad, idx_wmajor)


def _edge_mlp_body(xp_ref, nb_ref, w_ref, b_ref, o_ref, *, c, rows):
    center = xp_ref[:, :c]
    parts = []
    for k in range(KNN):
        parts.append(center)
        parts.append(nb_ref[k][:, :c] - center)
    e = jnp.concatenate(
        [jnp.concatenate(parts[2 * k:2 * k + 2], axis=1)
         for k in range(KNN)], axis=0)
    hmat = jnp.dot(e, w_ref[...], preferred_element_type=jnp.float32)
    h = hmat[:rows]
    for k in range(1, KNN):
        h = jnp.maximum(h, hmat[k * rows:(k + 1) * rows])
    o_ref[...] = h + b_ref[...]


def _edge_mlp(xpad, nb, w, bias, c, *, rows=128):
    m = xpad.shape[0]
    cout = w.shape[1]
    body = functools.partial(_edge_mlp_body, c=c, rows=rows)
    return pl.pallas_call(
        body,
        grid=(m // rows,),
        in_specs=[
            pl.BlockSpec((rows, CPAD), lambda r: (r, 0)),
            pl.BlockSpec((KNN, rows, CPAD), lambda r: (0, r, 0)),
            pl.BlockSpec(w.shape, lambda r: (0, 0)),
            pl.BlockSpec((1, cout), lambda r: (0, 0)),
        ],
        out_specs=pl.BlockSpec((rows, cout), lambda r: (r, 0)),
        out_shape=jax.ShapeDtypeStruct((m, cout), jnp.float32),
    )(xpad, nb, w, bias[None, :])


def _head_body(h_ref, wf1_ref, bf1_ref, wf2_ref, bf2_ref, wf3_ref, bf3_ref,
               o_ref):
    g = jnp.max(h_ref[...], axis=1)
    z = jnp.dot(g, wf1_ref[...], preferred_element_type=jnp.float32)
    z = jnp.maximum(z + bf1_ref[...], 0.0)
    z = jnp.dot(z, wf2_ref[...], preferred_element_type=jnp.float32)
    z = jnp.maximum(z + bf2_ref[...], 0.0)
    z = jnp.dot(z, wf3_ref[...], preferred_element_type=jnp.float32)
    o_ref[...] = z + bf3_ref[...]


def _head(h, wf1, bf1, wf2, bf2, wf3, bf3):
    bsz = h.shape[0]
    return pl.pallas_call(
        _head_body,
        out_shape=jax.ShapeDtypeStruct((bsz, wf3.shape[1]), jnp.float32),
    )(h, wf1, bf1[None, :], wf2, bf2[None, :], wf3, bf3[None, :])


def _edge_conv(h, w, b):
    bsz, n, c = h.shape
    m = bsz * n
    cout = w.shape[1]
    idx = _knn(h)
    nw = 32
    idx_wmajor = idx.reshape(nw, m // nw, KNN)
    idx_wmajor = jnp.swapaxes(idx_wmajor, 1, 2).reshape(-1)
    hflat = h.reshape(m, c)
    xpad = hflat if c == CPAD else jnp.pad(hflat, ((0, 0), (0, CPAD - c)))
    nb = _gather_rows(xpad, idx_wmajor)
    out = _edge_mlp(xpad, nb, w, b, c)
    return out.reshape(bsz, n, cout)


def kernel(x, W1, b1, W2, b2, W3, b3, Wf1, bf1, Wf2, bf2, Wf3, bf3):
    h1 = _edge_conv(x, W1, b1)
    h2 = _edge_conv(h1, W2, b2)
    h3 = _edge_conv(h2, W3, b3)
    return _head(h3, Wf1, bf1, Wf2, bf2, Wf3, bf3)


# speedup vs baseline: 16.7589x; 1.4474x over previous
import functools

import jax
import jax.numpy as jnp
from jax import lax
from jax.experimental import pallas as pl
from jax.experimental.pallas import tpu as pltpu
from jax.experimental.pallas import tpu_sc as plsc

KNN = 20
CPAD = 128


def _knn_body(x_rows_ref, xt_ref, idx_ref, *, n, k, rows):
    b = pl.program_id(0)
    xr = x_rows_ref[0]
    xt = xt_ref[0]
    sq_r = jnp.sum(xr * xr, axis=1, keepdims=True)
    sq_a = jnp.sum(xt * xt, axis=0, keepdims=True)
    inner = jnp.dot(xr, xt, preferred_element_type=jnp.float32)
    dist = (sq_r - 2.0 * inner) + sq_a

    iota_f = lax.broadcasted_iota(jnp.int32, (rows, n), 1).astype(jnp.float32)
    cur = dist
    cols = []
    for _ in range(k):
        m = jnp.min(cur, axis=1, keepdims=True)
        eqm = cur == m
        amin = jnp.min(jnp.where(eqm, iota_f, 4096.0), axis=1,
                       keepdims=True)
        cur = jnp.where(eqm, jnp.inf, cur)
        cols.append(amin)
    idx = jnp.concatenate(cols, axis=1).astype(jnp.int32)
    idx_ref[0] = idx + b * n


def _knn(x, *, rows=256):
    bsz, n, c = x.shape
    xt = jnp.swapaxes(x, 1, 2)
    body = functools.partial(_knn_body, n=n, k=KNN, rows=rows)
    return pl.pallas_call(
        body,
        grid=(bsz, n // rows),
        in_specs=[
            pl.BlockSpec((1, rows, c), lambda b, r: (b, r, 0)),
            pl.BlockSpec((1, c, n), lambda b, r: (b, 0, 0)),
        ],
        out_specs=pl.BlockSpec((1, rows, KNN), lambda b, r: (b, r, 0)),
        out_shape=jax.ShapeDtypeStruct((bsz, n, KNN), jnp.int32),
    )(x, xt)


def _gather_rows(xpad, idx_wmajor):
    m = xpad.shape[0]
    info = plsc.get_sparse_core_info()
    nw = info.num_cores * info.num_subcores
    pts_w = m // nw
    half = pts_w // 2
    mesh = plsc.VectorSubcoreMesh(core_axis_name="c", subcore_axis_name="s")

    @functools.partial(
        pl.kernel, mesh=mesh,
        out_type=jax.ShapeDtypeStruct((KNN, m, CPAD), jnp.float32),
        scratch_types=[
            pltpu.VMEM((KNN * pts_w,), jnp.int32),
            pltpu.VMEM((half, CPAD), jnp.float32),
            pltpu.VMEM((half, CPAD), jnp.float32),
            pltpu.SemaphoreType.DMA,
            pltpu.SemaphoreType.DMA,
        ],
    )
    def k(x_hbm, idx_hbm, nb_hbm, idx_v, buf0, buf1, sem0, sem1):
        wid = lax.axis_index("s") * info.num_cores + lax.axis_index("c")
        base = wid * pts_w
        pltpu.sync_copy(idx_hbm.at[pl.ds(wid * (KNN * pts_w), KNN * pts_w)],
                        idx_v)
        bufs = ((buf0, sem0), (buf1, sem1))

        def fire(t, buf, sem):
            off = t * half
            for j in range(half // 128):
                pltpu.async_copy(
                    x_hbm.at[idx_v.at[pl.ds(off + j * 128, 128)]],
                    buf.at[pl.ds(j * 128, 128), :], sem)

        fire(0, buf0, sem0)
        fire(1, buf1, sem1)

        def body(i, carry):
            for b in range(2):
                buf, sem = bufs[b]
                for j in range(half // 128):
                    pltpu.make_async_copy(
                        x_hbm.at[idx_v.at[pl.ds(0, 128)]],
                        buf.at[pl.ds(j * 128, 128), :], sem).wait()
                pltpu.sync_copy(buf,
                                nb_hbm.at[i, pl.ds(base + b * half, half), :])

                @pl.when(i < KNN - 1)
                def _():
                    fire(2 * i + b + 2, buf, sem)

            return carry

        lax.fori_loop(0, KNN, body, 0)

    return k(xpad, idx_wmajor)


def _edge_mlp_body(xp_ref, nb_ref, w_ref, b_ref, o_ref, *, c, rows):
    center = xp_ref[:, :c]
    parts = []
    for k in range(KNN):
        parts.append(center)
        parts.append(nb_ref[k][:, :c] - center)
    e = jnp.concatenate(
        [jnp.concatenate(parts[2 * k:2 * k + 2], axis=1)
         for k in range(KNN)], axis=0)
    hmat = jnp.dot(e, w_ref[...], preferred_element_type=jnp.float32)
    h = hmat[:rows]
    for k in range(1, KNN):
        h = jnp.maximum(h, hmat[k * rows:(k + 1) * rows])
    o_ref[...] = h + b_ref[...]


def _edge_mlp(xpad, nb, w, bias, c, *, rows=128):
    m = xpad.shape[0]
    cout = w.shape[1]
    body = functools.partial(_edge_mlp_body, c=c, rows=rows)
    return pl.pallas_call(
        body,
        grid=(m // rows,),
        in_specs=[
            pl.BlockSpec((rows, CPAD), lambda r: (r, 0)),
            pl.BlockSpec((KNN, rows, CPAD), lambda r: (0, r, 0)),
            pl.BlockSpec(w.shape, lambda r: (0, 0)),
            pl.BlockSpec((1, cout), lambda r: (0, 0)),
        ],
        out_specs=pl.BlockSpec((rows, cout), lambda r: (r, 0)),
        out_shape=jax.ShapeDtypeStruct((m, cout), jnp.float32),
    )(xpad, nb, w, bias[None, :])


def _head_body(h_ref, wf1_ref, bf1_ref, wf2_ref, bf2_ref, wf3_ref, bf3_ref,
               o_ref):
    g = jnp.max(h_ref[...], axis=1)
    z = jnp.dot(g, wf1_ref[...], preferred_element_type=jnp.float32)
    z = jnp.maximum(z + bf1_ref[...], 0.0)
    z = jnp.dot(z, wf2_ref[...], preferred_element_type=jnp.float32)
    z = jnp.maximum(z + bf2_ref[...], 0.0)
    z = jnp.dot(z, wf3_ref[...], preferred_element_type=jnp.float32)
    o_ref[...] = z + bf3_ref[...]


def _head(h, wf1, bf1, wf2, bf2, wf3, bf3):
    bsz = h.shape[0]
    return pl.pallas_call(
        _head_body,
        out_shape=jax.ShapeDtypeStruct((bsz, wf3.shape[1]), jnp.float32),
    )(h, wf1, bf1[None, :], wf2, bf2[None, :], wf3, bf3[None, :])


def _prep(h):
    bsz, n, c = h.shape
    m = bsz * n
    idx = _knn(h)
    nw = 32
    idx_wmajor = idx.reshape(nw, m // nw, KNN)
    idx_wmajor = jnp.swapaxes(idx_wmajor, 1, 2).reshape(-1)
    hflat = h.reshape(m, c)
    xpad = hflat if c == CPAD else jnp.pad(hflat, ((0, 0), (0, CPAD - c)))
    return xpad, idx_wmajor


def kernel(x, W1, b1, W2, b2, W3, b3, Wf1, bf1, Wf2, bf2, Wf3, bf3):
    ha, hb = x[:4], x[4:]
    for w, b in ((W1, b1), (W2, b2), (W3, b3)):
        bsz, n, c = ha.shape
        cout = w.shape[1]
        xa, ia = _prep(ha)
        nba = _gather_rows(xa, ia)
        xb, ib = _prep(hb)
        nbb = _gather_rows(xb, ib)
        ha = _edge_mlp(xa, nba, w, b, c).reshape(bsz, n, cout)
        hb = _edge_mlp(xb, nbb, w, b, c).reshape(bsz, n, cout)
    h3 = jnp.concatenate([ha, hb], axis=0)
    return _head(h3, Wf1, bf1, Wf2, bf2, Wf3, bf3)
